# baseline (speedup 1.0000x reference)
import functools

import jax
import jax.numpy as jnp
from jax import lax
from jax.experimental import pallas as pl
from jax.experimental.pallas import tpu as pltpu
from jax.experimental.pallas import tpu_sc as plsc

N_CLASS = 256
LANES = 16
NUM_CORES = 2
NUM_SUBCORES = 16
RCHUNK = 8


def _one_hot_sc(x3d, B, T):
    n_chunks = N_CLASS // RCHUNK
    n_groups = T // LANES
    cbits = RCHUNK.bit_length() - 1

    mesh = plsc.VectorSubcoreMesh(core_axis_name="c", subcore_axis_name="s")

    @functools.partial(
        pl.kernel,
        out_type=jax.ShapeDtypeStruct((B, N_CLASS, T), jnp.float32),
        mesh=mesh,
        compiler_params=pltpu.CompilerParams(needs_layout_passes=False),
        scratch_types=[
            pltpu.VMEM((T,), jnp.int32),
            pltpu.VMEM((RCHUNK, T), jnp.float32),
            pltpu.VMEM((RCHUNK, T), jnp.float32),
            pltpu.SemaphoreType.DMA,
            pltpu.SemaphoreType.DMA,
            pltpu.SemaphoreType.DMA,
        ],
    )
    def body(x_hbm, out_hbm, x_v, buf0, buf1, sem0, sem1, semx):
        b = lax.axis_index("s") * NUM_CORES + lax.axis_index("c")
        xcp = pltpu.make_async_copy(x_hbm.at[b, 0], x_v, semx)
        xcp.start()

        ones = jnp.full((LANES,), 1.0, jnp.float32)
        zeros = jnp.zeros((LANES,), jnp.float32)
        iota16 = lax.iota(jnp.int32, LANES)
        bufs = (buf0, buf1)
        sems = (sem0, sem1)

        def zero_cols(g, c):
            for r in range(RCHUNK):
                buf0[r, pl.ds(g * LANES, LANES)] = zeros
                buf1[r, pl.ds(g * LANES, LANES)] = zeros
            return c

        lax.fori_loop(0, n_groups, zero_cols, 0)
        xcp.wait()

        def scan(buf, i, i_unset):
            def group(g, c):
                for u in range(2):
                    gg = g * 2 + u
                    xv = x_v[pl.ds(gg * LANES, LANES)]
                    row = xv & (RCHUNK - 1)
                    col = iota16 + gg * LANES
                    grp = lax.shift_right_logical(xv, cbits)
                    if i_unset is not None:
                        plsc.store_scatter(buf, [row, col], zeros, mask=grp == i_unset)
                    plsc.store_scatter(buf, [row, col], ones, mask=grp == i)
                return c

            lax.fori_loop(0, n_groups // 2, group, 0)

        def start_out(buf, i, sem):
            cp = pltpu.make_async_copy(
                buf, out_hbm.at[b, pl.ds(i * RCHUNK, RCHUNK), :], sem
            )
            cp.start()
            return cp

        scan(buf0, 0, None)
        start_out(buf0, 0, sem0)
        scan(buf1, 1, None)
        start_out(buf1, 1, sem1)

        def pair(p, c):
            for k in range(2):
                i = 2 * p + k
                buf, sem = bufs[k], sems[k]
                pltpu.make_async_copy(
                    buf, out_hbm.at[b, pl.ds((i - 2) * RCHUNK, RCHUNK), :], sem
                ).wait()
                scan(buf, i, i - 2)
                start_out(buf, i, sem)
            return c

        lax.fori_loop(1, n_chunks // 2, pair, 0)

        for k in range(2):
            i = n_chunks - 2 + k
            pltpu.make_async_copy(
                bufs[k], out_hbm.at[b, pl.ds(i * RCHUNK, RCHUNK), :], sems[k]
            ).wait()

    return body(x3d)


def kernel(x):
    B = x.shape[0]
    T = x.shape[-1]
    if x.dtype != jnp.int32:
        x = x.astype(jnp.int32)
    return _one_hot_sc(x, B, T)


# speedup vs baseline: 1.0239x; 1.0239x over previous
import functools

import jax
import jax.numpy as jnp
from jax import lax
from jax.experimental import pallas as pl
from jax.experimental.pallas import tpu as pltpu
from jax.experimental.pallas import tpu_sc as plsc

N_CLASS = 256
LANES = 16
NUM_CORES = 2
NUM_SUBCORES = 16
RCHUNK = 8


def _one_hot_sc(x3d, B, T):
    n_chunks = N_CLASS // RCHUNK
    n_groups = T // LANES
    cbits = RCHUNK.bit_length() - 1

    mesh = plsc.VectorSubcoreMesh(core_axis_name="c", subcore_axis_name="s")

    @functools.partial(
        pl.kernel,
        out_type=jax.ShapeDtypeStruct((B, N_CLASS, T), jnp.float32),
        mesh=mesh,
        compiler_params=pltpu.CompilerParams(needs_layout_passes=False),
        scratch_types=[
            pltpu.VMEM((T,), jnp.int32),
            pltpu.VMEM((RCHUNK, T), jnp.float32),
            pltpu.VMEM((RCHUNK, T), jnp.float32),
            pltpu.SemaphoreType.DMA,
            pltpu.SemaphoreType.DMA,
            pltpu.SemaphoreType.DMA,
        ],
    )
    def body(x_hbm, out_hbm, x_v, buf0, buf1, sem0, sem1, semx):
        b = lax.axis_index("s") * NUM_CORES + lax.axis_index("c")
        xcp = pltpu.make_async_copy(x_hbm.at[b, 0], x_v, semx)
        xcp.start()

        ones = jnp.full((LANES,), 1.0, jnp.float32)
        zeros = jnp.zeros((LANES,), jnp.float32)
        iota16 = lax.iota(jnp.int32, LANES)
        bufs = (buf0, buf1)
        sems = (sem0, sem1)

        def zero_cols(g, c):
            for r in range(RCHUNK):
                buf0[r, pl.ds(g * LANES, LANES)] = zeros
                buf1[r, pl.ds(g * LANES, LANES)] = zeros
            return c

        lax.fori_loop(0, n_groups, zero_cols, 0)
        xcp.wait()

        def scan(buf, i, i_unset):
            def group(g, c):
                xv = x_v[pl.ds(g * LANES, LANES)]
                row = xv & (RCHUNK - 1)
                col = iota16 + g * LANES
                grp = lax.shift_right_logical(xv, cbits)
                if i_unset is not None:
                    plsc.store_scatter(buf, [row, col], zeros, mask=grp == i_unset)
                plsc.store_scatter(buf, [row, col], ones, mask=grp == i)
                return c

            lax.fori_loop(0, n_groups, group, 0)

        def start_out(buf, i, sem):
            cp = pltpu.make_async_copy(
                buf, out_hbm.at[b, pl.ds(i * RCHUNK, RCHUNK), :], sem
            )
            cp.start()
            return cp

        scan(buf0, 0, None)
        start_out(buf0, 0, sem0)
        scan(buf1, 1, None)
        start_out(buf1, 1, sem1)

        def pair(p, c):
            for k in range(2):
                i = 2 * p + k
                buf, sem = bufs[k], sems[k]
                pltpu.make_async_copy(
                    buf, out_hbm.at[b, pl.ds((i - 2) * RCHUNK, RCHUNK), :], sem
                ).wait()
                scan(buf, i, i - 2)
                start_out(buf, i, sem)
            return c

        lax.fori_loop(1, n_chunks // 2, pair, 0)

        for k in range(2):
            i = n_chunks - 2 + k
            pltpu.make_async_copy(
                bufs[k], out_hbm.at[b, pl.ds(i * RCHUNK, RCHUNK), :], sems[k]
            ).wait()

    return body(x3d)


def kernel(x):
    B = x.shape[0]
    T = x.shape[-1]
    if x.dtype != jnp.int32:
        x = x.astype(jnp.int32)
    return _one_hot_sc(x, B, T)
